# baseline (speedup 1.0000x reference)
import functools

import jax
import jax.numpy as jnp
from jax import lax
from jax.experimental import pallas as pl
from jax.experimental.pallas import tpu as pltpu
from jax.experimental.pallas import tpu_sc as plsc

N = 10000
F_IN = 128
NHID = 64
NCLS = 40
NCLSP = 48
E = 320000

NC = 2
NS = 16
NW = NC * NS
CHUNK = 128
CPT = 2 * (-(-E // (2 * NW * CHUNK)))
E_PAD = CPT * NW * CHUNK
N_PAD = 10240
ROWS_PER_TILE = N_PAD // NS
R_TC = 2048

_MESH = dict(core_axis_name="c", subcore_axis_name="s", num_cores=NC,
             num_subcores=NS)


@functools.cache
def _make_edge_agg(F, sp_chunks):
    mesh = plsc.VectorSubcoreMesh(**_MESH)

    scratch = [
        pltpu.VMEM((CPT, CHUNK), jnp.int32),
        pltpu.VMEM((CPT, CHUNK), jnp.int32),
        [pltpu.VMEM((CHUNK, F), jnp.float32) for _ in range(2)],
        pltpu.VMEM_SHARED((N_PAD, F), jnp.float32),
        pltpu.VMEM_SHARED((N_PAD, F), jnp.float32) if sp_chunks else None,
        [pltpu.SemaphoreType.DMA for _ in range(2)],
    ]

    @functools.partial(
        pl.kernel,
        out_type=jax.ShapeDtypeStruct((NC, N_PAD, F), jnp.float32),
        mesh=mesh,
        compiler_params=pltpu.CompilerParams(use_tc_tiling_on_sc=False),
        scratch_types=[s for s in scratch if s is not None],
    )
    def edge_agg(src_hbm, dst_hbm, table_hbm, zeros_hbm, out_hbm,
                 srcv, dstv, bufs, acc, *rest):
        if sp_chunks:
            tbl, gsem = rest
        else:
            gsem, = rest
            tbl = None
        cid = lax.axis_index("c")
        sid = lax.axis_index("s")
        w = cid * NS + sid
        sl = pl.ds(sid * ROWS_PER_TILE, ROWS_PER_TILE)
        pltpu.sync_copy(src_hbm.at[w], srcv)
        pltpu.sync_copy(dst_hbm.at[w], dstv)
        pltpu.sync_copy(zeros_hbm, acc.at[sl])
        if sp_chunks:
            pltpu.sync_copy(table_hbm.at[sl], tbl.at[sl])
        plsc.subcore_barrier()

        def start_gather(jj, buf, sem):
            if sp_chunks:
                @pl.when(jj < sp_chunks)
                def _():
                    pltpu.async_copy(tbl.at[srcv.at[jj]], buf, sem)

                @pl.when(jj >= sp_chunks)
                def _():
                    pltpu.async_copy(table_hbm.at[srcv.at[jj]], buf, sem)
            else:
                pltpu.async_copy(table_hbm.at[srcv.at[jj]], buf, sem)

        start_gather(0, bufs[0], gsem[0])

        @pl.loop(0, CPT // 2)
        def _(g):
            j0 = g * 2
            pltpu.make_async_copy(table_hbm.at[srcv.at[j0]], bufs[0],
                                  gsem[0]).wait()
            start_gather(j0 + 1, bufs[1], gsem[1])
            pltpu.sync_copy(bufs[0], acc.at[dstv.at[j0]], add=True)
            pltpu.make_async_copy(table_hbm.at[srcv.at[j0 + 1]], bufs[1],
                                  gsem[1]).wait()

            @pl.when(j0 + 2 < CPT)
            def _():
                start_gather(j0 + 2, bufs[0], gsem[0])

            pltpu.sync_copy(bufs[1], acc.at[dstv.at[j0 + 1]], add=True)

        plsc.subcore_barrier()
        pltpu.sync_copy(acc.at[sl], out_hbm.at[cid, sl])

    return edge_agg


@functools.cache
def _make_deg():
    mesh = plsc.VectorSubcoreMesh(**_MESH)

    @functools.partial(
        pl.kernel,
        out_type=jax.ShapeDtypeStruct((NC, N_PAD, 16), jnp.float32),
        mesh=mesh,
        compiler_params=pltpu.CompilerParams(use_tc_tiling_on_sc=False),
        scratch_types=[
            pltpu.VMEM((CPT, CHUNK), jnp.int32),
            pltpu.VMEM((CHUNK, 16), jnp.float32),
            pltpu.VMEM_SHARED((N_PAD, 16), jnp.float32),
            pltpu.SemaphoreType.DMA,
            pltpu.SemaphoreType.DMA,
        ],
    )
    def deg(dst_hbm, zeros_hbm, out_hbm, dstv, buf, acc, isem, ssem):
        cid = lax.axis_index("c")
        sid = lax.axis_index("s")
        w = cid * NS + sid
        sl = pl.ds(sid * ROWS_PER_TILE, ROWS_PER_TILE)
        c1 = pltpu.async_copy(dst_hbm.at[w], dstv, isem)
        c2 = pltpu.async_copy(zeros_hbm, acc.at[sl], ssem)
        one = jnp.ones((16,), jnp.float32)

        @pl.loop(0, CHUNK)
        def _(r):
            buf[r] = one

        c1.wait()
        c2.wait()
        plsc.subcore_barrier()
        @pl.loop(0, CPT // 8)
        def _(g):
            for b in range(8):
                pltpu.async_copy(buf, acc.at[dstv.at[g * 8 + b]], ssem,
                                 add=True)
            for b in range(8):
                pltpu.make_async_copy(buf,
                                      acc.at[dstv.at[g * 8 + b]],
                                      ssem).wait()

        plsc.subcore_barrier()
        pltpu.sync_copy(acc.at[sl], out_hbm.at[cid, sl])

    return deg


def _t1a_body(x_ref, w_ref, h_ref):
    h_ref[...] = jnp.dot(x_ref[...], w_ref[...],
                         preferred_element_type=jnp.float32)


def _t1a(x, W1):
    grid = (N_PAD // R_TC,)
    return pl.pallas_call(
        _t1a_body,
        grid=grid,
        in_specs=[
            pl.BlockSpec((R_TC, F_IN), lambda i: (i, 0)),
            pl.BlockSpec((F_IN, NHID), lambda i: (0, 0)),
        ],
        out_specs=pl.BlockSpec((R_TC, NHID), lambda i: (i, 0)),
        out_shape=jax.ShapeDtypeStruct((N_PAD, NHID), jnp.float32),
    )(x, W1)


def _t1b_body(h_ref, degp_ref, hs_ref, dinv_ref):
    degsum = degp_ref[0] + degp_ref[1]
    dinv = lax.rsqrt(degsum + 1.0)
    dinv_ref[...] = dinv
    hs_ref[...] = h_ref[...] * dinv[:, :1]


def _t1b(h1, degp):
    grid = (N_PAD // R_TC,)
    return pl.pallas_call(
        _t1b_body,
        grid=grid,
        in_specs=[
            pl.BlockSpec((R_TC, NHID), lambda i: (i, 0)),
            pl.BlockSpec((NC, R_TC, 16), lambda i: (0, i, 0)),
        ],
        out_specs=[
            pl.BlockSpec((R_TC, NHID), lambda i: (i, 0)),
            pl.BlockSpec((R_TC, 16), lambda i: (i, 0)),
        ],
        out_shape=[
            jax.ShapeDtypeStruct((N_PAD, NHID), jnp.float32),
            jax.ShapeDtypeStruct((N_PAD, 16), jnp.float32),
        ],
    )(h1, degp)


def _t2_body(agg_ref, hs_ref, dinv_ref, w_ref, b_ref, out_ref):
    a = agg_ref[0] + agg_ref[1] + hs_ref[...]
    dinv = dinv_ref[:, :1]
    out1 = jnp.maximum(dinv * a + b_ref[...], 0.0)
    h2 = jnp.dot(out1, w_ref[...], preferred_element_type=jnp.float32)
    out_ref[...] = h2 * dinv


def _t2(agg1, hs1, dinv, W2p, b1r):
    grid = (N_PAD // R_TC,)
    return pl.pallas_call(
        _t2_body,
        grid=grid,
        in_specs=[
            pl.BlockSpec((NC, R_TC, NHID), lambda i: (0, i, 0)),
            pl.BlockSpec((R_TC, NHID), lambda i: (i, 0)),
            pl.BlockSpec((R_TC, 16), lambda i: (i, 0)),
            pl.BlockSpec((NHID, NCLSP), lambda i: (0, 0)),
            pl.BlockSpec((1, NHID), lambda i: (0, 0)),
        ],
        out_specs=pl.BlockSpec((R_TC, NCLSP), lambda i: (i, 0)),
        out_shape=jax.ShapeDtypeStruct((N_PAD, NCLSP), jnp.float32),
    )(agg1, hs1, dinv, W2p, b1r)


def _t3_body(agg_ref, hs_ref, dinv_ref, b_ref, out_ref):
    a = agg_ref[0] + agg_ref[1] + hs_ref[...]
    logits = dinv_ref[:, :1] * a + b_ref[...]
    col = lax.broadcasted_iota(jnp.int32, (R_TC, NCLSP), 1)
    logits = jnp.where(col < NCLS, logits, -jnp.inf)
    m = jnp.max(logits, axis=1, keepdims=True)
    ex = jnp.exp(logits - m)
    s = jnp.sum(ex, axis=1, keepdims=True)
    res = (logits - m) - jnp.log(s)
    out_ref[...] = res[:, :NCLS]


def _t3(agg2, hs2, dinv, b2r):
    grid = (N_PAD // R_TC,)
    return pl.pallas_call(
        _t3_body,
        grid=grid,
        in_specs=[
            pl.BlockSpec((NC, R_TC, NCLSP), lambda i: (0, i, 0)),
            pl.BlockSpec((R_TC, NCLSP), lambda i: (i, 0)),
            pl.BlockSpec((R_TC, 16), lambda i: (i, 0)),
            pl.BlockSpec((1, NCLSP), lambda i: (0, 0)),
        ],
        out_specs=pl.BlockSpec((R_TC, NCLS), lambda i: (i, 0)),
        out_shape=jax.ShapeDtypeStruct((N, NCLS), jnp.float32),
    )(agg2, hs2, dinv, b2r)


def kernel(x, edge_index, W1, b1, W2, b2, sigma1, sigma2):
    del sigma1, sigma2
    src = edge_index[0].astype(jnp.int32)
    dst = edge_index[1].astype(jnp.int32)
    npad_e = E_PAD - E
    pad_idx = N + (jnp.arange(npad_e, dtype=jnp.int32) % (N_PAD - N))
    srcp = jnp.concatenate([src, pad_idx]).reshape(NW, CPT, CHUNK)
    dstp = jnp.concatenate([dst, pad_idx]).reshape(NW, CPT, CHUNK)
    W2p = jnp.pad(W2, ((0, 0), (0, NCLSP - NCLS)))
    b1r = b1.reshape(1, NHID)
    b2r = jnp.pad(b2, (0, NCLSP - NCLS)).reshape(1, NCLSP)
    zeros16 = jnp.zeros((ROWS_PER_TILE, 16), jnp.float32)
    zeros64 = jnp.zeros((ROWS_PER_TILE, NHID), jnp.float32)
    zeros48 = jnp.zeros((ROWS_PER_TILE, NCLSP), jnp.float32)
    degp = _make_deg()(dstp, zeros16)
    h1 = _t1a(x, W1)
    hs1, dinv = _t1b(h1, degp)
    agg1 = _make_edge_agg(NHID, 30)(srcp, dstp, hs1, zeros64)
    hs2 = _t2(agg1, hs1, dinv, W2p, b1r)
    agg2 = _make_edge_agg(NCLSP, 30)(srcp, dstp, hs2, zeros48)
    return _t3(agg2, hs2, dinv, b2r)


# speedup vs baseline: 1.0105x; 1.0105x over previous
import functools

import jax
import jax.numpy as jnp
from jax import lax
from jax.experimental import pallas as pl
from jax.experimental.pallas import tpu as pltpu
from jax.experimental.pallas import tpu_sc as plsc

N = 10000
F_IN = 128
NHID = 64
NCLS = 40
NCLSP = 64
E = 320000

NC = 2
NS = 16
NW = NC * NS
CHUNK = 128
CPT = 2 * (-(-E // (2 * NW * CHUNK)))
E_PAD = CPT * NW * CHUNK
N_PAD = 10240
ROWS_PER_TILE = N_PAD // NS
R_TC = 2048

_MESH = dict(core_axis_name="c", subcore_axis_name="s", num_cores=NC,
             num_subcores=NS)


CPT2 = E_PAD // NS // CHUNK


@functools.cache
def _make_edge_agg(FH):
    mesh = plsc.VectorSubcoreMesh(**_MESH)

    @functools.partial(
        pl.kernel,
        out_type=jax.ShapeDtypeStruct((NC, N_PAD, FH), jnp.float32),
        mesh=mesh,
        compiler_params=pltpu.CompilerParams(use_tc_tiling_on_sc=False),
        scratch_types=[
            pltpu.VMEM((CPT2, CHUNK), jnp.int32),
            pltpu.VMEM((CPT2, CHUNK), jnp.int32),
            [pltpu.VMEM((CHUNK, FH), jnp.float32) for _ in range(2)],
            pltpu.VMEM_SHARED((N_PAD, FH), jnp.float32),
            pltpu.VMEM_SHARED((N_PAD, FH), jnp.float32),
            [pltpu.SemaphoreType.DMA for _ in range(2)],
        ],
    )
    def edge_agg(src_hbm, dst_hbm, table_hbm, zeros_hbm, out_hbm,
                 srcv, dstv, bufs, acc, tbl, gsem):
        cid = lax.axis_index("c")
        sid = lax.axis_index("s")
        sl = pl.ds(sid * ROWS_PER_TILE, ROWS_PER_TILE)
        pltpu.sync_copy(src_hbm.at[sid], srcv)
        pltpu.sync_copy(dst_hbm.at[sid], dstv)
        pltpu.sync_copy(zeros_hbm, acc.at[sl])
        pltpu.sync_copy(table_hbm.at[cid, sl], tbl.at[sl])
        plsc.subcore_barrier()
        pltpu.async_copy(tbl.at[srcv.at[0]], bufs[0], gsem[0])

        @pl.loop(0, CPT2 // 2)
        def _(g):
            j0 = g * 2
            pltpu.make_async_copy(tbl.at[srcv.at[j0]], bufs[0],
                                  gsem[0]).wait()
            pltpu.async_copy(tbl.at[srcv.at[j0 + 1]], bufs[1], gsem[1])
            pltpu.sync_copy(bufs[0], acc.at[dstv.at[j0]], add=True)
            pltpu.make_async_copy(tbl.at[srcv.at[j0 + 1]], bufs[1],
                                  gsem[1]).wait()

            @pl.when(j0 + 2 < CPT2)
            def _():
                pltpu.async_copy(tbl.at[srcv.at[j0 + 2]], bufs[0], gsem[0])

            pltpu.sync_copy(bufs[1], acc.at[dstv.at[j0 + 1]], add=True)

        plsc.subcore_barrier()
        pltpu.sync_copy(acc.at[sl], out_hbm.at[cid, sl])

    return edge_agg


@functools.cache
def _make_deg():
    mesh = plsc.VectorSubcoreMesh(**_MESH)

    @functools.partial(
        pl.kernel,
        out_type=jax.ShapeDtypeStruct((NC, N_PAD, 16), jnp.float32),
        mesh=mesh,
        compiler_params=pltpu.CompilerParams(use_tc_tiling_on_sc=False),
        scratch_types=[
            pltpu.VMEM((CPT, CHUNK), jnp.int32),
            pltpu.VMEM((CHUNK, 16), jnp.float32),
            pltpu.VMEM_SHARED((N_PAD, 16), jnp.float32),
            pltpu.SemaphoreType.DMA,
            pltpu.SemaphoreType.DMA,
        ],
    )
    def deg(dst_hbm, zeros_hbm, out_hbm, dstv, buf, acc, isem, ssem):
        cid = lax.axis_index("c")
        sid = lax.axis_index("s")
        w = cid * NS + sid
        sl = pl.ds(sid * ROWS_PER_TILE, ROWS_PER_TILE)
        c1 = pltpu.async_copy(
            dst_hbm.at[w // 2, pl.ds((w % 2) * CPT, CPT)], dstv, isem)
        c2 = pltpu.async_copy(zeros_hbm, acc.at[sl], ssem)
        one = jnp.ones((16,), jnp.float32)

        @pl.loop(0, CHUNK)
        def _(r):
            buf[r] = one

        c1.wait()
        c2.wait()
        plsc.subcore_barrier()
        @pl.loop(0, CPT // 8)
        def _(g):
            for b in range(8):
                pltpu.async_copy(buf, acc.at[dstv.at[g * 8 + b]], ssem,
                                 add=True)
            for b in range(8):
                pltpu.make_async_copy(buf,
                                      acc.at[dstv.at[g * 8 + b]],
                                      ssem).wait()

        plsc.subcore_barrier()
        pltpu.sync_copy(acc.at[sl], out_hbm.at[cid, sl])

    return deg


def _t1a_body(x_ref, w_ref, h_ref):
    h_ref[...] = jnp.dot(x_ref[...], w_ref[...],
                         preferred_element_type=jnp.float32)


def _t1a(x, W1):
    grid = (N_PAD // R_TC,)
    return pl.pallas_call(
        _t1a_body,
        grid=grid,
        in_specs=[
            pl.BlockSpec((R_TC, F_IN), lambda i: (i, 0)),
            pl.BlockSpec((F_IN, NHID), lambda i: (0, 0)),
        ],
        out_specs=pl.BlockSpec((R_TC, NHID), lambda i: (i, 0)),
        out_shape=jax.ShapeDtypeStruct((N_PAD, NHID), jnp.float32),
    )(x, W1)


def _t1b_body(h_ref, degp_ref, hs_ref, dinv_ref):
    degsum = degp_ref[0] + degp_ref[1]
    dinv = lax.rsqrt(degsum + 1.0)
    dinv_ref[...] = dinv
    hs = h_ref[...] * dinv[:, :1]
    hs_ref[0] = hs[:, :NHID // 2]
    hs_ref[1] = hs[:, NHID // 2:]


def _t1b(h1, degp):
    grid = (N_PAD // R_TC,)
    return pl.pallas_call(
        _t1b_body,
        grid=grid,
        in_specs=[
            pl.BlockSpec((R_TC, NHID), lambda i: (i, 0)),
            pl.BlockSpec((NC, R_TC, 16), lambda i: (0, i, 0)),
        ],
        out_specs=[
            pl.BlockSpec((NC, R_TC, NHID // 2), lambda i: (0, i, 0)),
            pl.BlockSpec((R_TC, 16), lambda i: (i, 0)),
        ],
        out_shape=[
            jax.ShapeDtypeStruct((NC, N_PAD, NHID // 2), jnp.float32),
            jax.ShapeDtypeStruct((N_PAD, 16), jnp.float32),
        ],
    )(h1, degp)


def _t2_body(agg_ref, hs_ref, dinv_ref, w_ref, b_ref, out_ref):
    a = jnp.concatenate([agg_ref[0] + hs_ref[0], agg_ref[1] + hs_ref[1]],
                        axis=1)
    dinv = dinv_ref[:, :1]
    out1 = jnp.maximum(dinv * a + b_ref[...], 0.0)
    h2 = jnp.dot(out1, w_ref[...], preferred_element_type=jnp.float32)
    hs2 = h2 * dinv
    out_ref[0] = hs2[:, :NCLSP // 2]
    out_ref[1] = hs2[:, NCLSP // 2:]


def _t2(agg1, hs1s, dinv, W2p, b1r):
    grid = (N_PAD // R_TC,)
    return pl.pallas_call(
        _t2_body,
        grid=grid,
        in_specs=[
            pl.BlockSpec((NC, R_TC, NHID // 2), lambda i: (0, i, 0)),
            pl.BlockSpec((NC, R_TC, NHID // 2), lambda i: (0, i, 0)),
            pl.BlockSpec((R_TC, 16), lambda i: (i, 0)),
            pl.BlockSpec((NHID, NCLSP), lambda i: (0, 0)),
            pl.BlockSpec((1, NHID), lambda i: (0, 0)),
        ],
        out_specs=pl.BlockSpec((NC, R_TC, NCLSP // 2), lambda i: (0, i, 0)),
        out_shape=jax.ShapeDtypeStruct((NC, N_PAD, NCLSP // 2), jnp.float32),
    )(agg1, hs1s, dinv, W2p, b1r)


def _t3_body(agg_ref, hs_ref, dinv_ref, b_ref, out_ref):
    a = jnp.concatenate([agg_ref[0] + hs_ref[0], agg_ref[1] + hs_ref[1]],
                        axis=1)
    logits = dinv_ref[:, :1] * a + b_ref[...]
    col = lax.broadcasted_iota(jnp.int32, (R_TC, NCLSP), 1)
    logits = jnp.where(col < NCLS, logits, -jnp.inf)
    m = jnp.max(logits, axis=1, keepdims=True)
    ex = jnp.exp(logits - m)
    s = jnp.sum(ex, axis=1, keepdims=True)
    res = (logits - m) - jnp.log(s)
    out_ref[...] = res[:, :NCLS]


def _t3(agg2, hs2, dinv, b2r):
    grid = (N_PAD // R_TC,)
    return pl.pallas_call(
        _t3_body,
        grid=grid,
        in_specs=[
            pl.BlockSpec((NC, R_TC, NCLSP // 2), lambda i: (0, i, 0)),
            pl.BlockSpec((NC, R_TC, NCLSP // 2), lambda i: (0, i, 0)),
            pl.BlockSpec((R_TC, 16), lambda i: (i, 0)),
            pl.BlockSpec((1, NCLSP), lambda i: (0, 0)),
        ],
        out_specs=pl.BlockSpec((R_TC, NCLS), lambda i: (i, 0)),
        out_shape=jax.ShapeDtypeStruct((N, NCLS), jnp.float32),
    )(agg2, hs2, dinv, b2r)


def kernel(x, edge_index, W1, b1, W2, b2, sigma1, sigma2):
    del sigma1, sigma2
    src = edge_index[0].astype(jnp.int32)
    dst = edge_index[1].astype(jnp.int32)
    npad_e = E_PAD - E
    pad_idx = N + (jnp.arange(npad_e, dtype=jnp.int32) % (N_PAD - N))
    srcp = jnp.concatenate([src, pad_idx]).reshape(NS, CPT2, CHUNK)
    dstp = jnp.concatenate([dst, pad_idx]).reshape(NS, CPT2, CHUNK)
    W2p = jnp.pad(W2, ((0, 0), (0, NCLSP - NCLS)))
    b1r = b1.reshape(1, NHID)
    b2r = jnp.pad(b2, (0, NCLSP - NCLS)).reshape(1, NCLSP)
    zeros16 = jnp.zeros((ROWS_PER_TILE, 16), jnp.float32)
    zeros32 = jnp.zeros((ROWS_PER_TILE, NHID // 2), jnp.float32)
    degp = _make_deg()(dstp, zeros16)
    h1 = _t1a(x, W1)
    hs1s, dinv = _t1b(h1, degp)
    agg1 = _make_edge_agg(NHID // 2)(srcp, dstp, hs1s, zeros32)
    hs2s = _t2(agg1, hs1s, dinv, W2p, b1r)
    agg2 = _make_edge_agg(NCLSP // 2)(srcp, dstp, hs2s, zeros32)
    return _t3(agg2, hs2s, dinv, b2r)


# speedup vs baseline: 1.1909x; 1.1785x over previous
import functools

import jax
import jax.numpy as jnp
from jax import lax
from jax.experimental import pallas as pl
from jax.experimental.pallas import tpu as pltpu
from jax.experimental.pallas import tpu_sc as plsc

N = 10000
F_IN = 128
NHID = 64
NCLS = 40
NCLSP = 48
E = 320000

NC = 2
NS = 16
NW = NC * NS
CHUNK = 128
CPT = 2 * (-(-E // (2 * NW * CHUNK)))
E_PAD = CPT * NW * CHUNK
N_PAD = 10240
ROWS_PER_TILE = N_PAD // NS
R_TC = 2048

_MESH = dict(core_axis_name="c", subcore_axis_name="s", num_cores=NC,
             num_subcores=NS)


CPT2 = E_PAD // NS // CHUNK


@functools.cache
def _make_edge_agg(F):
    mesh = plsc.VectorSubcoreMesh(**_MESH)

    @functools.partial(
        pl.kernel,
        out_type=jax.ShapeDtypeStruct((NC, N_PAD, 128), jnp.float32),
        mesh=mesh,
        compiler_params=pltpu.CompilerParams(use_tc_tiling_on_sc=False),
        scratch_types=[
            pltpu.VMEM((CPT, CHUNK), jnp.int32),
            pltpu.VMEM((CPT, CHUNK), jnp.int32),
            [pltpu.VMEM((CHUNK, F), jnp.float32) for _ in range(2)],
            pltpu.VMEM_SHARED((N_PAD, F), jnp.float32),
            pltpu.VMEM_SHARED((N_PAD, F), jnp.float32),
            [pltpu.SemaphoreType.DMA for _ in range(2)],
        ],
    )
    def edge_agg(src_hbm, dst_hbm, table_hbm, zeros_hbm, out_hbm,
                 srcv, dstv, bufs, acc, tbl, gsem):
        cid = lax.axis_index("c")
        sid = lax.axis_index("s")
        w = cid * NS + sid
        sl = pl.ds(sid * ROWS_PER_TILE, ROWS_PER_TILE)
        pltpu.sync_copy(src_hbm.at[w // 2, pl.ds((w % 2) * CPT, CPT)], srcv)
        pltpu.sync_copy(dst_hbm.at[w // 2, pl.ds((w % 2) * CPT, CPT)], dstv)
        pltpu.sync_copy(zeros_hbm, acc.at[sl])
        pltpu.sync_copy(table_hbm.at[sl], tbl.at[sl])
        plsc.subcore_barrier()
        pltpu.async_copy(tbl.at[srcv.at[0]], bufs[0], gsem[0])

        @pl.loop(0, CPT // 2)
        def _(g):
            j0 = g * 2
            pltpu.make_async_copy(tbl.at[srcv.at[j0]], bufs[0],
                                  gsem[0]).wait()
            pltpu.async_copy(tbl.at[srcv.at[j0 + 1]], bufs[1], gsem[1])
            pltpu.sync_copy(bufs[0], acc.at[dstv.at[j0]], add=True)
            pltpu.make_async_copy(tbl.at[srcv.at[j0 + 1]], bufs[1],
                                  gsem[1]).wait()

            @pl.when(j0 + 2 < CPT)
            def _():
                pltpu.async_copy(tbl.at[srcv.at[j0 + 2]], bufs[0], gsem[0])

            pltpu.sync_copy(bufs[1], acc.at[dstv.at[j0 + 1]], add=True)

        plsc.subcore_barrier()
        pltpu.sync_copy(acc.at[sl], out_hbm.at[cid, sl, pl.ds(0, F)])

    return edge_agg


@functools.cache
def _make_deg():
    mesh = plsc.VectorSubcoreMesh(**_MESH)

    @functools.partial(
        pl.kernel,
        out_type=jax.ShapeDtypeStruct((NC, N_PAD, 128), jnp.float32),
        mesh=mesh,
        compiler_params=pltpu.CompilerParams(use_tc_tiling_on_sc=False),
        scratch_types=[
            pltpu.VMEM((CPT, CHUNK), jnp.int32),
            pltpu.VMEM((CHUNK, 16), jnp.float32),
            pltpu.VMEM_SHARED((N_PAD, 16), jnp.float32),
            pltpu.SemaphoreType.DMA,
            pltpu.SemaphoreType.DMA,
        ],
    )
    def deg(dst_hbm, zeros_hbm, out_hbm, dstv, buf, acc, isem, ssem):
        cid = lax.axis_index("c")
        sid = lax.axis_index("s")
        w = cid * NS + sid
        sl = pl.ds(sid * ROWS_PER_TILE, ROWS_PER_TILE)
        c1 = pltpu.async_copy(
            dst_hbm.at[w // 2, pl.ds((w % 2) * CPT, CPT)], dstv, isem)
        c2 = pltpu.async_copy(zeros_hbm, acc.at[sl], ssem)
        one = jnp.ones((16,), jnp.float32)

        @pl.loop(0, CHUNK)
        def _(r):
            buf[r] = one

        c1.wait()
        c2.wait()
        plsc.subcore_barrier()
        @pl.loop(0, CPT // 8)
        def _(g):
            for b in range(8):
                pltpu.async_copy(buf, acc.at[dstv.at[g * 8 + b]], ssem,
                                 add=True)
            for b in range(8):
                pltpu.make_async_copy(buf,
                                      acc.at[dstv.at[g * 8 + b]],
                                      ssem).wait()

        plsc.subcore_barrier()
        pltpu.sync_copy(acc.at[sl], out_hbm.at[cid, sl, pl.ds(0, 16)])

    return deg


def _t1a_body(x_ref, w_ref, h_ref):
    h_ref[...] = jnp.dot(x_ref[...], w_ref[...],
                         preferred_element_type=jnp.float32)


def _t1a(x, W1):
    grid = (N_PAD // R_TC,)
    return pl.pallas_call(
        _t1a_body,
        grid=grid,
        in_specs=[
            pl.BlockSpec((R_TC, F_IN), lambda i: (i, 0)),
            pl.BlockSpec((F_IN, NHID), lambda i: (0, 0)),
        ],
        out_specs=pl.BlockSpec((R_TC, NHID), lambda i: (i, 0)),
        out_shape=jax.ShapeDtypeStruct((N_PAD, NHID), jnp.float32),
    )(x, W1)


def _t1b_body(h_ref, degp_ref, hs_ref, dinv_ref):
    degsum = degp_ref[0, :, :16] + degp_ref[1, :, :16]
    dinv = lax.rsqrt(degsum + 1.0)
    dinv_ref[...] = dinv
    hs_ref[...] = h_ref[...] * dinv[:, :1]


def _t1b(h1, degp):
    grid = (N_PAD // R_TC,)
    return pl.pallas_call(
        _t1b_body,
        grid=grid,
        in_specs=[
            pl.BlockSpec((R_TC, NHID), lambda i: (i, 0)),
            pl.BlockSpec((NC, R_TC, 128), lambda i: (0, i, 0)),
        ],
        out_specs=[
            pl.BlockSpec((R_TC, NHID), lambda i: (i, 0)),
            pl.BlockSpec((R_TC, 16), lambda i: (i, 0)),
        ],
        out_shape=[
            jax.ShapeDtypeStruct((N_PAD, NHID), jnp.float32),
            jax.ShapeDtypeStruct((N_PAD, 16), jnp.float32),
        ],
    )(h1, degp)


def _t2_body(agg_ref, hs_ref, dinv_ref, w_ref, b_ref, out_ref):
    a = agg_ref[0, :, :NHID] + agg_ref[1, :, :NHID] + hs_ref[...]
    dinv = dinv_ref[:, :1]
    out1 = jnp.maximum(dinv * a + b_ref[...], 0.0)
    h2 = jnp.dot(out1, w_ref[...], preferred_element_type=jnp.float32)
    out_ref[...] = h2 * dinv


def _t2(agg1, hs1, dinv, W2p, b1r):
    grid = (N_PAD // R_TC,)
    return pl.pallas_call(
        _t2_body,
        grid=grid,
        in_specs=[
            pl.BlockSpec((NC, R_TC, 128), lambda i: (0, i, 0)),
            pl.BlockSpec((R_TC, NHID), lambda i: (i, 0)),
            pl.BlockSpec((R_TC, 16), lambda i: (i, 0)),
            pl.BlockSpec((NHID, NCLSP), lambda i: (0, 0)),
            pl.BlockSpec((1, NHID), lambda i: (0, 0)),
        ],
        out_specs=pl.BlockSpec((R_TC, NCLSP), lambda i: (i, 0)),
        out_shape=jax.ShapeDtypeStruct((N_PAD, NCLSP), jnp.float32),
    )(agg1, hs1, dinv, W2p, b1r)


def _t3_body(agg_ref, hs_ref, dinv_ref, b_ref, out_ref):
    a = agg_ref[0, :, :NCLSP] + agg_ref[1, :, :NCLSP] + hs_ref[...]
    logits = dinv_ref[:, :1] * a + b_ref[...]
    col = lax.broadcasted_iota(jnp.int32, (R_TC, NCLSP), 1)
    logits = jnp.where(col < NCLS, logits, -jnp.inf)
    m = jnp.max(logits, axis=1, keepdims=True)
    ex = jnp.exp(logits - m)
    s = jnp.sum(ex, axis=1, keepdims=True)
    res = (logits - m) - jnp.log(s)
    out_ref[...] = res[:, :NCLS]


def _t3(agg2, hs2, dinv, b2r):
    grid = (N_PAD // R_TC,)
    return pl.pallas_call(
        _t3_body,
        grid=grid,
        in_specs=[
            pl.BlockSpec((NC, R_TC, 128), lambda i: (0, i, 0)),
            pl.BlockSpec((R_TC, NCLSP), lambda i: (i, 0)),
            pl.BlockSpec((R_TC, 16), lambda i: (i, 0)),
            pl.BlockSpec((1, NCLSP), lambda i: (0, 0)),
        ],
        out_specs=pl.BlockSpec((R_TC, NCLS), lambda i: (i, 0)),
        out_shape=jax.ShapeDtypeStruct((N, NCLS), jnp.float32),
    )(agg2, hs2, dinv, b2r)


def kernel(x, edge_index, W1, b1, W2, b2, sigma1, sigma2):
    del sigma1, sigma2
    src = edge_index[0].astype(jnp.int32)
    dst = edge_index[1].astype(jnp.int32)
    npad_e = E_PAD - E
    pad_idx = N + (jnp.arange(npad_e, dtype=jnp.int32) % (N_PAD - N))
    srcp = jnp.concatenate([src, pad_idx]).reshape(NS, CPT2, CHUNK)
    dstp = jnp.concatenate([dst, pad_idx]).reshape(NS, CPT2, CHUNK)
    W2p = jnp.pad(W2, ((0, 0), (0, NCLSP - NCLS)))
    b1r = b1.reshape(1, NHID)
    b2r = jnp.pad(b2, (0, NCLSP - NCLS)).reshape(1, NCLSP)
    zeros16 = jnp.zeros((ROWS_PER_TILE, 16), jnp.float32)
    zeros64 = jnp.zeros((ROWS_PER_TILE, NHID), jnp.float32)
    zeros48 = jnp.zeros((ROWS_PER_TILE, NCLSP), jnp.float32)
    degp = _make_deg()(dstp, zeros16)
    h1 = _t1a(x, W1)
    hs1, dinv = _t1b(h1, degp)
    agg1 = _make_edge_agg(NHID)(srcp, dstp, hs1, zeros64)
    hs2 = _t2(agg1, hs1, dinv, W2p, b1r)
    agg2 = _make_edge_agg(NCLSP)(srcp, dstp, hs2, zeros48)
    return _t3(agg2, hs2, dinv, b2r)
